# Initial kernel scaffold; baseline (speedup 1.0000x reference)
import jax
import jax.numpy as jnp
from jax.experimental import pallas as pl


def kernel(atom_fea, nbr_fea, nbr_fea_idx, crystal_atom_idx, emb_w, fc_w_0, fc_b_0, bn1_g_0, bn1_b_0, bn2_g_0, bn2_b_0, fc_w_1, fc_b_1, bn1_g_1, bn1_b_1, bn2_g_1, bn2_b_1, fc_w_2, fc_b_2, bn1_g_2, bn1_b_2, bn2_g_2, bn2_b_2, bil_w, bil_b, fc1_w, fc1_b, fcaf_w, fcaf_b):
    raise NotImplementedError("write your pallas kernel here")


# speedup vs baseline: 2.3702x; 2.3702x over previous
import functools

import jax
import jax.numpy as jnp
from jax import lax
from jax.experimental import pallas as pl
from jax.experimental.pallas import tpu as pltpu
from jax.experimental.pallas import tpu_sc as plsc

N_ATOMS = 10000
M = 16
ORIG = 92
NBRF = 41
AF = 64
TWO_AF = 2 * AF
B = 8
NC = 128
N_EDGE = N_ATOMS * M

A_BLK = 400
GRID = N_ATOMS // A_BLK


def _sc_gather(table, idx_flat, window):
    n = idx_flat.shape[0]
    d = table.shape[1]
    idx2 = idx_flat.reshape(1, n)
    mesh = plsc.VectorSubcoreMesh(core_axis_name="c", subcore_axis_name="s")

    @functools.partial(
        pl.kernel,
        out_type=jax.ShapeDtypeStruct((n, d), table.dtype),
        mesh=mesh,
    )
    def k(x_hbm, i_hbm, o_hbm):
        def body(i_vmem, o_vmem):
            pltpu.sync_copy(x_hbm.at[i_vmem.at[0]], o_vmem)

        pltpu.emit_pipeline(
            body,
            grid=(n // window,),
            in_specs=[pl.BlockSpec((1, window), lambda i: (0, i))],
            out_specs=[pl.BlockSpec((window, d), lambda i: (i, 0))],
            core_axis_name=("c", "s"),
            dimension_semantics=(pltpu.PARALLEL,),
        )(i_hbm, o_hbm)

    return k(table, idx2)


def _embed_body(a_ref, w_ref, out_ref):
    xe = jnp.dot(a_ref[...], w_ref[...], preferred_element_type=jnp.float32)
    out_ref[...] = jnp.concatenate(
        [xe, jnp.zeros((xe.shape[0], TWO_AF - AF), jnp.float32)], axis=1)


def _prep_body(x_ref, ws_ref, wn_ref, s_ref, zn_ref):
    x = x_ref[...][:, :AF]
    s_ref[...] = jnp.dot(x, ws_ref[...], preferred_element_type=jnp.float32)
    zn_ref[...] = jnp.dot(x, wn_ref[...], preferred_element_type=jnp.float32)


def _gated_block(s_blk, g_blk, nbr_blk, we_t, bias):
    a = s_blk.shape[0]
    ze = jnp.dot(nbr_blk.reshape(a * M, NBRF), we_t,
                 preferred_element_type=jnp.float32)
    return (g_blk.reshape(a, M, TWO_AF) + ze.reshape(a, M, TWO_AF)
            + s_blk[:, None, :] + bias[...][None])


def _p1_body(s_ref, g_ref, nbr_ref, we_ref, b_ref, acc_ref):
    gated = _gated_block(s_ref[...], g_ref[...], nbr_ref[...],
                         we_ref[...], b_ref)
    part = jnp.concatenate(
        [jnp.sum(gated, axis=(0, 1))[None, :],
         jnp.sum(gated * gated, axis=(0, 1))[None, :]], axis=0)

    @pl.when(pl.program_id(0) == 0)
    def _():
        acc_ref[...] = jnp.zeros_like(acc_ref)

    acc_ref[...] += part


def _p2_body(s_ref, g_ref, nbr_ref, we_ref, b_ref,
             acc1_ref, g1_ref, b1_ref, ns_ref, acc2_ref):
    cnt = jnp.float32(N_EDGE)
    mean = acc1_ref[...][0:1, :] / cnt
    var = acc1_ref[...][1:2, :] / cnt - mean * mean
    scale = lax.rsqrt(var + 1e-5) * g1_ref[...]
    shift = b1_ref[...] - mean * scale
    gated = _gated_block(s_ref[...], g_ref[...], nbr_ref[...],
                         we_ref[...], b_ref)
    normed = gated * scale[None] + shift[None]
    filt = jax.nn.sigmoid(normed[..., :AF])
    core = jax.nn.softplus(normed[..., AF:])
    ns = jnp.sum(filt * core, axis=1)
    ns_ref[...] = ns
    part = jnp.concatenate(
        [jnp.sum(ns, axis=0)[None, :],
         jnp.sum(ns * ns, axis=0)[None, :]], axis=0)

    @pl.when(pl.program_id(0) == 0)
    def _():
        acc2_ref[...] = jnp.zeros_like(acc2_ref)

    acc2_ref[...] += part


def _p3_body(x_ref, ns_ref, acc2_ref, g2_ref, b2_ref, out_ref):
    cnt = jnp.float32(N_ATOMS)
    mean = acc2_ref[...][0:1, :] / cnt
    var = acc2_ref[...][1:2, :] / cnt - mean * mean
    scale = lax.rsqrt(var + 1e-5) * g2_ref[...]
    shift = b2_ref[...] - mean * scale
    y = ns_ref[...] * scale + shift
    xn = jax.nn.softplus(x_ref[...][:, :AF] + y)
    out_ref[...] = jnp.concatenate(
        [xn, jnp.zeros((xn.shape[0], TWO_AF - AF), jnp.float32)], axis=1)


def _dec_body(af_ref, bilt_ref, bilb_ref, fc1t_ref, fc1b_ref,
              fcaft_ref, fcafb_ref, ep_ref, feat_ref):
    af = af_ref[...][:, :AF]
    cols = []
    for o in range(6):
        t = jnp.dot(af, bilt_ref[o], preferred_element_type=jnp.float32)
        cols.append(jnp.sum(t * af, axis=1, keepdims=True))
    q = jnp.concatenate(cols, axis=1) + bilb_ref[...]
    p = jnp.dot(q, fc1t_ref[...], preferred_element_type=jnp.float32) \
        + fc1b_ref[...]
    mx = jnp.max(p, axis=1, keepdims=True)
    lsm = p - mx - jnp.log(jnp.sum(jnp.exp(p - mx), axis=1, keepdims=True))
    ep_ref[...] = jnp.broadcast_to(
        lsm[None, :, :], (NC, NC, 6)).reshape(1, NC * NC, 6)
    feat_ref[...] = (jnp.dot(af, fcaft_ref[...],
                             preferred_element_type=jnp.float32)
                     + fcafb_ref[...])


def _conv(x_pad, g, nbr_fea, s, we_t, bias, g1, b1, g2, b2):
    row2 = lambda v: v.reshape(1, -1)
    wspecs = [
        pl.BlockSpec((NBRF, TWO_AF), lambda i: (0, 0)),
        pl.BlockSpec((1, TWO_AF), lambda i: (0, 0)),
    ]
    dspecs = [
        pl.BlockSpec((A_BLK, TWO_AF), lambda i: (i, 0)),
        pl.BlockSpec((A_BLK * M, TWO_AF), lambda i: (i, 0)),
        pl.BlockSpec((A_BLK, M, NBRF), lambda i: (i, 0, 0)),
    ]
    acc1 = pl.pallas_call(
        _p1_body,
        grid=(GRID,),
        in_specs=dspecs + wspecs,
        out_specs=pl.BlockSpec((2, TWO_AF), lambda i: (0, 0)),
        out_shape=jax.ShapeDtypeStruct((2, TWO_AF), jnp.float32),
    )(s, g, nbr_fea, we_t, row2(bias))

    ns, acc2 = pl.pallas_call(
        _p2_body,
        grid=(GRID,),
        in_specs=dspecs + wspecs + [
            pl.BlockSpec((2, TWO_AF), lambda i: (0, 0)),
            pl.BlockSpec((1, TWO_AF), lambda i: (0, 0)),
            pl.BlockSpec((1, TWO_AF), lambda i: (0, 0)),
        ],
        out_specs=[
            pl.BlockSpec((A_BLK, AF), lambda i: (i, 0)),
            pl.BlockSpec((2, AF), lambda i: (0, 0)),
        ],
        out_shape=[
            jax.ShapeDtypeStruct((N_ATOMS, AF), jnp.float32),
            jax.ShapeDtypeStruct((2, AF), jnp.float32),
        ],
    )(s, g, nbr_fea, we_t, row2(bias), acc1, row2(g1), row2(b1))

    return pl.pallas_call(
        _p3_body,
        grid=(1,),
        in_specs=[
            pl.BlockSpec((N_ATOMS, TWO_AF), lambda i: (0, 0)),
            pl.BlockSpec((N_ATOMS, AF), lambda i: (0, 0)),
            pl.BlockSpec((2, AF), lambda i: (0, 0)),
            pl.BlockSpec((1, AF), lambda i: (0, 0)),
            pl.BlockSpec((1, AF), lambda i: (0, 0)),
        ],
        out_specs=pl.BlockSpec((N_ATOMS, TWO_AF), lambda i: (0, 0)),
        out_shape=jax.ShapeDtypeStruct((N_ATOMS, TWO_AF), jnp.float32),
    )(x_pad, ns, acc2, row2(g2), row2(b2))


def kernel(atom_fea, nbr_fea, nbr_fea_idx, crystal_atom_idx, emb_w,
           fc_w_0, fc_b_0, bn1_g_0, bn1_b_0, bn2_g_0, bn2_b_0,
           fc_w_1, fc_b_1, bn1_g_1, bn1_b_1, bn2_g_1, bn2_b_1,
           fc_w_2, fc_b_2, bn1_g_2, bn1_b_2, bn2_g_2, bn2_b_2,
           bil_w, bil_b, fc1_w, fc1_b, fcaf_w, fcaf_b):
    nbr_idx_flat = nbr_fea_idx.reshape(-1).astype(jnp.int32)
    cry_flat = crystal_atom_idx.reshape(-1).astype(jnp.int32)

    x_pad = pl.pallas_call(
        _embed_body,
        grid=(1,),
        in_specs=[
            pl.BlockSpec((N_ATOMS, ORIG), lambda i: (0, 0)),
            pl.BlockSpec((ORIG, AF), lambda i: (0, 0)),
        ],
        out_specs=pl.BlockSpec((N_ATOMS, TWO_AF), lambda i: (0, 0)),
        out_shape=jax.ShapeDtypeStruct((N_ATOMS, TWO_AF), jnp.float32),
    )(atom_fea, emb_w.T)

    convs = [
        (fc_w_0, fc_b_0, bn1_g_0, bn1_b_0, bn2_g_0, bn2_b_0),
        (fc_w_1, fc_b_1, bn1_g_1, bn1_b_1, bn2_g_1, bn2_b_1),
        (fc_w_2, fc_b_2, bn1_g_2, bn1_b_2, bn2_g_2, bn2_b_2),
    ]
    for fc_w, fc_b, g1, b1, g2, b2 in convs:
        ws_t = fc_w[:, :AF].T
        wn_t = fc_w[:, AF:TWO_AF].T
        we_t = fc_w[:, TWO_AF:].T
        s, zn = pl.pallas_call(
            _prep_body,
            grid=(1,),
            in_specs=[
                pl.BlockSpec((N_ATOMS, TWO_AF), lambda i: (0, 0)),
                pl.BlockSpec((AF, TWO_AF), lambda i: (0, 0)),
                pl.BlockSpec((AF, TWO_AF), lambda i: (0, 0)),
            ],
            out_specs=[
                pl.BlockSpec((N_ATOMS, TWO_AF), lambda i: (0, 0)),
                pl.BlockSpec((N_ATOMS, TWO_AF), lambda i: (0, 0)),
            ],
            out_shape=[
                jax.ShapeDtypeStruct((N_ATOMS, TWO_AF), jnp.float32),
                jax.ShapeDtypeStruct((N_ATOMS, TWO_AF), jnp.float32),
            ],
        )(x_pad, ws_t, wn_t)
        g = _sc_gather(zn, nbr_idx_flat, window=128)
        x_pad = _conv(x_pad, g, nbr_fea, s, we_t, fc_b, g1, b1, g2, b2)

    af = _sc_gather(x_pad, cry_flat, window=128)

    ep, feat = pl.pallas_call(
        _dec_body,
        grid=(B,),
        in_specs=[
            pl.BlockSpec((NC, TWO_AF), lambda i: (i, 0)),
            pl.BlockSpec((6, AF, AF), lambda i: (0, 0, 0)),
            pl.BlockSpec((1, 6), lambda i: (0, 0)),
            pl.BlockSpec((6, 6), lambda i: (0, 0)),
            pl.BlockSpec((1, 6), lambda i: (0, 0)),
            pl.BlockSpec((AF, ORIG), lambda i: (0, 0)),
            pl.BlockSpec((1, ORIG), lambda i: (0, 0)),
        ],
        out_specs=[
            pl.BlockSpec((1, NC * NC, 6), lambda i: (i, 0, 0)),
            pl.BlockSpec((NC, ORIG), lambda i: (i, 0)),
        ],
        out_shape=[
            jax.ShapeDtypeStruct((B, NC * NC, 6), jnp.float32),
            jax.ShapeDtypeStruct((B * NC, ORIG), jnp.float32),
        ],
    )(af, jnp.swapaxes(bil_w, 1, 2), bil_b.reshape(1, 6),
      fc1_w.T, fc1_b.reshape(1, 6), fcaf_w.T, fcaf_b.reshape(1, ORIG))

    return (ep, feat.reshape(B, NC, ORIG))


# speedup vs baseline: 2.6645x; 1.1242x over previous
import functools

import jax
import jax.numpy as jnp
from jax import lax
from jax.experimental import pallas as pl
from jax.experimental.pallas import tpu as pltpu
from jax.experimental.pallas import tpu_sc as plsc

N_ATOMS = 10000
M = 16
ORIG = 92
NBRF = 41
AF = 64
TWO_AF = 2 * AF
B = 8
NC = 128
N_EDGE = N_ATOMS * M

A_BLK = 400
GRID = N_ATOMS // A_BLK


def _sc_gather(table, idx_flat, window):
    n = idx_flat.shape[0]
    d = table.shape[1]
    idx2 = idx_flat.reshape(1, n)
    mesh = plsc.VectorSubcoreMesh(core_axis_name="c", subcore_axis_name="s")

    @functools.partial(
        pl.kernel,
        out_type=jax.ShapeDtypeStruct((n, d), table.dtype),
        mesh=mesh,
    )
    def k(x_hbm, i_hbm, o_hbm):
        def body(i_vmem, o_vmem):
            pltpu.sync_copy(x_hbm.at[i_vmem.at[0]], o_vmem)

        pltpu.emit_pipeline(
            body,
            grid=(n // window,),
            in_specs=[pl.BlockSpec((1, window), lambda i: (0, i))],
            out_specs=[pl.BlockSpec((window, d), lambda i: (i, 0))],
            core_axis_name=("c", "s"),
            dimension_semantics=(pltpu.PARALLEL,),
        )(i_hbm, o_hbm)

    return k(table, idx2)


def _embed_body(a_ref, w_ref, out_ref):
    xe = jnp.dot(a_ref[...], w_ref[...], preferred_element_type=jnp.float32)
    out_ref[...] = jnp.concatenate(
        [xe, jnp.zeros((xe.shape[0], TWO_AF - AF), jnp.float32)], axis=1)


def _prep_body(x_ref, ws_ref, wn_ref, s_ref, zn_ref):
    x = x_ref[...][:, :AF]
    s_ref[...] = jnp.dot(x, ws_ref[...], preferred_element_type=jnp.float32)
    zn_ref[...] = jnp.dot(x, wn_ref[...], preferred_element_type=jnp.float32)


def _p12_body(s_ref, g_ref, nbr_ref, we_ref, b_ref, g1_ref, b1_ref,
              ns_ref, acc2_ref, gated_ref, stat_ref):
    ph = pl.program_id(0)
    i = pl.program_id(1)
    rows = A_BLK * M

    @pl.when(ph == 0)
    def _():
        ze = jnp.dot(nbr_ref[...].reshape(rows, NBRF), we_ref[...],
                     preferred_element_type=jnp.float32)
        gated = (g_ref[...].reshape(A_BLK, M, TWO_AF)
                 + ze.reshape(A_BLK, M, TWO_AF)
                 + s_ref[...][:, None, :] + b_ref[...][None])
        gated_ref[pl.ds(i * rows, rows), :] = \
            gated.reshape(rows, TWO_AF).astype(jnp.bfloat16)
        part = jnp.concatenate(
            [jnp.sum(gated, axis=(0, 1))[None, :],
             jnp.sum(gated * gated, axis=(0, 1))[None, :]], axis=0)
        prev = jnp.where(i == 0, jnp.zeros_like(part), stat_ref[...])
        stat_ref[...] = prev + part

    @pl.when(ph == 1)
    def _():
        cnt = jnp.float32(N_EDGE)
        mean = stat_ref[...][0:1, :] / cnt
        var = stat_ref[...][1:2, :] / cnt - mean * mean
        scale = lax.rsqrt(var + 1e-5) * g1_ref[...]
        shift = b1_ref[...] - mean * scale
        gated = gated_ref[pl.ds(i * rows, rows), :].astype(jnp.float32)
        normed = (gated * scale + shift).reshape(A_BLK, M, TWO_AF)
        filt = jax.nn.sigmoid(normed[..., :AF])
        core = jax.nn.softplus(normed[..., AF:])
        ns = jnp.sum(filt * core, axis=1)
        ns_ref[...] = ns
        part = jnp.concatenate(
            [jnp.sum(ns, axis=0)[None, :],
             jnp.sum(ns * ns, axis=0)[None, :]], axis=0)
        prev = jnp.where(i == 0, jnp.zeros_like(part), acc2_ref[...])
        acc2_ref[...] = prev + part


def _p3_body(x_ref, ns_ref, acc2_ref, g2_ref, b2_ref, out_ref):
    cnt = jnp.float32(N_ATOMS)
    mean = acc2_ref[...][0:1, :] / cnt
    var = acc2_ref[...][1:2, :] / cnt - mean * mean
    scale = lax.rsqrt(var + 1e-5) * g2_ref[...]
    shift = b2_ref[...] - mean * scale
    y = ns_ref[...] * scale + shift
    xn = jax.nn.softplus(x_ref[...][:, :AF] + y)
    out_ref[...] = jnp.concatenate(
        [xn, jnp.zeros((xn.shape[0], TWO_AF - AF), jnp.float32)], axis=1)


def _dec_body(af_ref, bilt_ref, bilb_ref, fc1_ref, fc1b_ref,
              fcafw_ref, fcafb_ref, ep_ref, feat_ref):
    for bi in range(B):
        af = af_ref[bi * NC:(bi + 1) * NC, :AF]
        rows = []
        for o in range(6):
            t = jnp.dot(af, bilt_ref[o], preferred_element_type=jnp.float32)
            rows.append(jnp.sum(t * af, axis=1)[None, :])
        qt = jnp.concatenate(rows, axis=0) + bilb_ref[...]
        pt = jnp.dot(fc1_ref[...], qt,
                     preferred_element_type=jnp.float32) + fc1b_ref[...]
        mx = jnp.max(pt, axis=0, keepdims=True)
        lsmt = pt - mx - jnp.log(jnp.sum(jnp.exp(pt - mx), axis=0,
                                         keepdims=True))
        ep_ref[:, bi, :] = jnp.tile(lsmt, (1, NC))
        ft = lax.dot_general(fcafw_ref[...], af,
                             (((1,), (1,)), ((), ())),
                             preferred_element_type=jnp.float32) \
            + fcafb_ref[...]
        feat_ref[:, bi, :] = ft


def _conv(x_pad, g, nbr_fea, s, we_t, bias, g1, b1, g2, b2):
    row2 = lambda v: v.reshape(1, -1)
    ns, acc2 = pl.pallas_call(
        _p12_body,
        grid=(2, GRID),
        in_specs=[
            pl.BlockSpec((A_BLK, TWO_AF),
                         lambda p, i: (jnp.where(p == 0, i, 0), 0)),
            pl.BlockSpec((A_BLK * M, TWO_AF),
                         lambda p, i: (jnp.where(p == 0, i, 0), 0)),
            pl.BlockSpec((A_BLK, M, NBRF),
                         lambda p, i: (jnp.where(p == 0, i, 0), 0, 0)),
            pl.BlockSpec((NBRF, TWO_AF), lambda p, i: (0, 0)),
            pl.BlockSpec((1, TWO_AF), lambda p, i: (0, 0)),
            pl.BlockSpec((1, TWO_AF), lambda p, i: (0, 0)),
            pl.BlockSpec((1, TWO_AF), lambda p, i: (0, 0)),
        ],
        out_specs=[
            pl.BlockSpec((A_BLK, AF),
                         lambda p, i: (jnp.where(p == 1, i, 0), 0)),
            pl.BlockSpec((2, AF), lambda p, i: (0, 0)),
        ],
        out_shape=[
            jax.ShapeDtypeStruct((N_ATOMS, AF), jnp.float32),
            jax.ShapeDtypeStruct((2, AF), jnp.float32),
        ],
        scratch_shapes=[
            pltpu.VMEM((N_EDGE, TWO_AF), jnp.bfloat16),
            pltpu.VMEM((2, TWO_AF), jnp.float32),
        ],
    )(s, g, nbr_fea, we_t, row2(bias), row2(g1), row2(b1))

    return pl.pallas_call(
        _p3_body,
        grid=(1,),
        in_specs=[
            pl.BlockSpec((N_ATOMS, TWO_AF), lambda i: (0, 0)),
            pl.BlockSpec((N_ATOMS, AF), lambda i: (0, 0)),
            pl.BlockSpec((2, AF), lambda i: (0, 0)),
            pl.BlockSpec((1, AF), lambda i: (0, 0)),
            pl.BlockSpec((1, AF), lambda i: (0, 0)),
        ],
        out_specs=pl.BlockSpec((N_ATOMS, TWO_AF), lambda i: (0, 0)),
        out_shape=jax.ShapeDtypeStruct((N_ATOMS, TWO_AF), jnp.float32),
    )(x_pad, ns, acc2, row2(g2), row2(b2))


def kernel(atom_fea, nbr_fea, nbr_fea_idx, crystal_atom_idx, emb_w,
           fc_w_0, fc_b_0, bn1_g_0, bn1_b_0, bn2_g_0, bn2_b_0,
           fc_w_1, fc_b_1, bn1_g_1, bn1_b_1, bn2_g_1, bn2_b_1,
           fc_w_2, fc_b_2, bn1_g_2, bn1_b_2, bn2_g_2, bn2_b_2,
           bil_w, bil_b, fc1_w, fc1_b, fcaf_w, fcaf_b):
    nbr_idx_flat = nbr_fea_idx.reshape(-1).astype(jnp.int32)
    cry_flat = crystal_atom_idx.reshape(-1).astype(jnp.int32)

    x_pad = pl.pallas_call(
        _embed_body,
        grid=(1,),
        in_specs=[
            pl.BlockSpec((N_ATOMS, ORIG), lambda i: (0, 0)),
            pl.BlockSpec((ORIG, AF), lambda i: (0, 0)),
        ],
        out_specs=pl.BlockSpec((N_ATOMS, TWO_AF), lambda i: (0, 0)),
        out_shape=jax.ShapeDtypeStruct((N_ATOMS, TWO_AF), jnp.float32),
    )(atom_fea, emb_w.T)

    convs = [
        (fc_w_0, fc_b_0, bn1_g_0, bn1_b_0, bn2_g_0, bn2_b_0),
        (fc_w_1, fc_b_1, bn1_g_1, bn1_b_1, bn2_g_1, bn2_b_1),
        (fc_w_2, fc_b_2, bn1_g_2, bn1_b_2, bn2_g_2, bn2_b_2),
    ]
    for fc_w, fc_b, g1, b1, g2, b2 in convs:
        ws_t = fc_w[:, :AF].T
        wn_t = fc_w[:, AF:TWO_AF].T
        we_t = fc_w[:, TWO_AF:].T
        s, zn = pl.pallas_call(
            _prep_body,
            grid=(1,),
            in_specs=[
                pl.BlockSpec((N_ATOMS, TWO_AF), lambda i: (0, 0)),
                pl.BlockSpec((AF, TWO_AF), lambda i: (0, 0)),
                pl.BlockSpec((AF, TWO_AF), lambda i: (0, 0)),
            ],
            out_specs=[
                pl.BlockSpec((N_ATOMS, TWO_AF), lambda i: (0, 0)),
                pl.BlockSpec((N_ATOMS, TWO_AF), lambda i: (0, 0)),
            ],
            out_shape=[
                jax.ShapeDtypeStruct((N_ATOMS, TWO_AF), jnp.float32),
                jax.ShapeDtypeStruct((N_ATOMS, TWO_AF), jnp.float32),
            ],
        )(x_pad, ws_t, wn_t)
        g = _sc_gather(zn, nbr_idx_flat, window=128)
        x_pad = _conv(x_pad, g, nbr_fea, s, we_t, fc_b, g1, b1, g2, b2)

    af = _sc_gather(x_pad, cry_flat, window=128)

    ep6, feat6 = pl.pallas_call(
        _dec_body,
        grid=(1,),
        in_specs=[
            pl.BlockSpec((B * NC, TWO_AF), lambda i: (0, 0)),
            pl.BlockSpec((6, AF, AF), lambda i: (0, 0, 0)),
            pl.BlockSpec((6, 1), lambda i: (0, 0)),
            pl.BlockSpec((6, 6), lambda i: (0, 0)),
            pl.BlockSpec((6, 1), lambda i: (0, 0)),
            pl.BlockSpec((ORIG, AF), lambda i: (0, 0)),
            pl.BlockSpec((ORIG, 1), lambda i: (0, 0)),
        ],
        out_specs=[
            pl.BlockSpec((6, B, NC * NC), lambda i: (0, 0, 0)),
            pl.BlockSpec((ORIG, B, NC), lambda i: (0, 0, 0)),
        ],
        out_shape=[
            jax.ShapeDtypeStruct((6, B, NC * NC), jnp.float32),
            jax.ShapeDtypeStruct((ORIG, B, NC), jnp.float32),
        ],
    )(af, jnp.swapaxes(bil_w, 1, 2), bil_b.reshape(6, 1),
      fc1_w, fc1_b.reshape(6, 1), fcaf_w, fcaf_b.reshape(ORIG, 1))

    return (jnp.transpose(ep6, (1, 2, 0)), jnp.transpose(feat6, (1, 2, 0)))


# speedup vs baseline: 2.8940x; 1.0861x over previous
import functools

import jax
import jax.numpy as jnp
from jax import lax
from jax.experimental import pallas as pl
from jax.experimental.pallas import tpu as pltpu
from jax.experimental.pallas import tpu_sc as plsc

N_ATOMS = 10000
M = 16
ORIG = 92
NBRF = 41
AF = 64
TWO_AF = 2 * AF
B = 8
NC = 128
N_EDGE = N_ATOMS * M

A_BLK = 512
GRID = (N_ATOMS + A_BLK - 1) // A_BLK


def _sc_gather(table, idx_flat, window):
    n = idx_flat.shape[0]
    d = table.shape[1]
    idx2 = idx_flat.reshape(1, n)
    mesh = plsc.VectorSubcoreMesh(core_axis_name="c", subcore_axis_name="s")

    @functools.partial(
        pl.kernel,
        out_type=jax.ShapeDtypeStruct((n, d), table.dtype),
        mesh=mesh,
    )
    def k(x_hbm, i_hbm, o_hbm):
        def body(i_vmem, o_vmem):
            pltpu.sync_copy(x_hbm.at[i_vmem.at[0]], o_vmem)

        pltpu.emit_pipeline(
            body,
            grid=(n // window,),
            in_specs=[pl.BlockSpec((1, window), lambda i: (0, i))],
            out_specs=[pl.BlockSpec((window, d), lambda i: (i, 0))],
            core_axis_name=("c", "s"),
            dimension_semantics=(pltpu.PARALLEL,),
        )(i_hbm, o_hbm)

    return k(table, idx2)


def _embed_body(a_ref, w_ref, out_ref):
    xe = lax.dot_general(a_ref[...], w_ref[...], (((0,), (1,)), ((), ())),
                         preferred_element_type=jnp.float32)
    out_ref[...] = jnp.concatenate(
        [xe, jnp.zeros((xe.shape[0], TWO_AF - AF), jnp.float32)], axis=1)


def _prep_body(x_ref, ws_ref, wn_ref, s_ref, zn_ref):
    x = x_ref[...][:, :AF]
    s_ref[...] = jnp.dot(x, ws_ref[...], preferred_element_type=jnp.float32)
    zn_ref[...] = jnp.dot(x, wn_ref[...], preferred_element_type=jnp.float32)


def _p12_body(s_ref, g_ref, nbr_ref, web_ref, b_ref, g1_ref, b1_ref,
              ns_ref, acc2_ref, gated_ref, stat_ref):
    ph = pl.program_id(0)
    i = pl.program_id(1)

    @pl.when(ph == 0)
    def _():
        ze_all = lax.dot_general(nbr_ref[...].astype(jnp.bfloat16),
                                 web_ref[...], (((0,), (0,)), ((), ())),
                                 preferred_element_type=jnp.float32)
        s_blk = s_ref[...] + b_ref[...]
        valid = (i * A_BLK + lax.broadcasted_iota(jnp.int32, (A_BLK, 1), 0)
                 ) < N_ATOMS
        ssum = jnp.zeros((1, TWO_AF), jnp.float32)
        ssq = jnp.zeros((1, TWO_AF), jnp.float32)
        for m in range(M):
            gated = (ze_all[:, m * TWO_AF:(m + 1) * TWO_AF]
                     + g_ref[m] + s_blk)
            gated_ref[pl.ds((i * M + m) * A_BLK, A_BLK), :] = \
                gated.astype(jnp.bfloat16)
            gmask = jnp.where(valid, gated, 0.0)
            ssum = ssum + jnp.sum(gmask, axis=0)[None, :]
            ssq = ssq + jnp.sum(gmask * gmask, axis=0)[None, :]
        part = jnp.concatenate([ssum, ssq], axis=0)
        prev = jnp.where(i == 0, jnp.zeros_like(part), stat_ref[...])
        stat_ref[...] = prev + part

    @pl.when(ph == 1)
    def _():
        cnt = jnp.float32(N_EDGE)
        mean = stat_ref[...][0:1, :] / cnt
        var = stat_ref[...][1:2, :] / cnt - mean * mean
        scale = lax.rsqrt(var + 1e-5) * g1_ref[...]
        shift = b1_ref[...] - mean * scale
        ns = jnp.zeros((A_BLK, AF), jnp.float32)
        for m in range(M):
            gated = gated_ref[pl.ds((i * M + m) * A_BLK, A_BLK),
                              :].astype(jnp.float32)
            normed = gated * scale + shift
            filt = jax.nn.sigmoid(normed[:, :AF])
            core = jax.nn.softplus(normed[:, AF:])
            ns = ns + filt * core
        ns_ref[...] = ns
        valid = (i * A_BLK + lax.broadcasted_iota(jnp.int32, (A_BLK, 1), 0)
                 ) < N_ATOMS
        nsm = jnp.where(valid, ns, 0.0)
        part = jnp.concatenate(
            [jnp.sum(nsm, axis=0)[None, :],
             jnp.sum(nsm * nsm, axis=0)[None, :]], axis=0)
        prev = jnp.where(i == 0, jnp.zeros_like(part), acc2_ref[...])
        acc2_ref[...] = prev + part


def _p3_body(x_ref, ns_ref, acc2_ref, g2_ref, b2_ref, out_ref):
    cnt = jnp.float32(N_ATOMS)
    mean = acc2_ref[...][0:1, :] / cnt
    var = acc2_ref[...][1:2, :] / cnt - mean * mean
    scale = lax.rsqrt(var + 1e-5) * g2_ref[...]
    shift = b2_ref[...] - mean * scale
    y = ns_ref[...] * scale + shift
    xn = jax.nn.softplus(x_ref[...][:, :AF] + y)
    out_ref[...] = jnp.concatenate(
        [xn, jnp.zeros((xn.shape[0], TWO_AF - AF), jnp.float32)], axis=1)


def _dec_body(af_ref, bilt_ref, bilb_ref, fc1_ref, fc1b_ref,
              fcafw_ref, fcafb_ref, ep_ref, feat_ref):
    for bi in range(B):
        af = af_ref[bi * NC:(bi + 1) * NC, :AF]
        rows = []
        for o in range(6):
            t = jnp.dot(af, bilt_ref[o], preferred_element_type=jnp.float32)
            rows.append(jnp.sum(t * af, axis=1)[None, :])
        qt = jnp.concatenate(rows, axis=0) + bilb_ref[...]
        pt = jnp.dot(fc1_ref[...], qt,
                     preferred_element_type=jnp.float32) + fc1b_ref[...]
        mx = jnp.max(pt, axis=0, keepdims=True)
        lsmt = pt - mx - jnp.log(jnp.sum(jnp.exp(pt - mx), axis=0,
                                         keepdims=True))
        ep_ref[:, bi, :] = jnp.tile(lsmt, (1, NC))
        ft = lax.dot_general(fcafw_ref[...], af,
                             (((1,), (1,)), ((), ())),
                             preferred_element_type=jnp.float32) \
            + fcafb_ref[...]
        feat_ref[:, bi, :] = ft


def _conv(x_pad, g3, nbr2, s, web, bias, g1, b1, g2, b2):
    row2 = lambda v: v.reshape(1, -1)
    ns, acc2 = pl.pallas_call(
        _p12_body,
        grid=(2, GRID),
        in_specs=[
            pl.BlockSpec((A_BLK, TWO_AF),
                         lambda p, i: (jnp.where(p == 0, i, 0), 0)),
            pl.BlockSpec((M, A_BLK, TWO_AF),
                         lambda p, i: (0, jnp.where(p == 0, i, 0), 0)),
            pl.BlockSpec((M * NBRF, A_BLK),
                         lambda p, i: (0, jnp.where(p == 0, i, 0))),
            pl.BlockSpec((M * NBRF, M * TWO_AF), lambda p, i: (0, 0)),
            pl.BlockSpec((1, TWO_AF), lambda p, i: (0, 0)),
            pl.BlockSpec((1, TWO_AF), lambda p, i: (0, 0)),
            pl.BlockSpec((1, TWO_AF), lambda p, i: (0, 0)),
        ],
        out_specs=[
            pl.BlockSpec((A_BLK, AF),
                         lambda p, i: (jnp.where(p == 1, i, 0), 0)),
            pl.BlockSpec((2, AF), lambda p, i: (0, 0)),
        ],
        out_shape=[
            jax.ShapeDtypeStruct((N_ATOMS, AF), jnp.float32),
            jax.ShapeDtypeStruct((2, AF), jnp.float32),
        ],
        scratch_shapes=[
            pltpu.VMEM((GRID * M * A_BLK, TWO_AF), jnp.bfloat16),
            pltpu.VMEM((2, TWO_AF), jnp.float32),
        ],
    )(s, g3, nbr2, web, row2(bias), row2(g1), row2(b1))

    return pl.pallas_call(
        _p3_body,
        grid=(1,),
        in_specs=[
            pl.BlockSpec((N_ATOMS, TWO_AF), lambda i: (0, 0)),
            pl.BlockSpec((N_ATOMS, AF), lambda i: (0, 0)),
            pl.BlockSpec((2, AF), lambda i: (0, 0)),
            pl.BlockSpec((1, AF), lambda i: (0, 0)),
            pl.BlockSpec((1, AF), lambda i: (0, 0)),
        ],
        out_specs=pl.BlockSpec((N_ATOMS, TWO_AF), lambda i: (0, 0)),
        out_shape=jax.ShapeDtypeStruct((N_ATOMS, TWO_AF), jnp.float32),
    )(x_pad, ns, acc2, row2(g2), row2(b2))


def kernel(atom_fea, nbr_fea, nbr_fea_idx, crystal_atom_idx, emb_w,
           fc_w_0, fc_b_0, bn1_g_0, bn1_b_0, bn2_g_0, bn2_b_0,
           fc_w_1, fc_b_1, bn1_g_1, bn1_b_1, bn2_g_1, bn2_b_1,
           fc_w_2, fc_b_2, bn1_g_2, bn1_b_2, bn2_g_2, bn2_b_2,
           bil_w, bil_b, fc1_w, fc1_b, fcaf_w, fcaf_b):
    nbr_idx_flat = jnp.transpose(nbr_fea_idx).reshape(-1).astype(jnp.int32)
    nbr2 = jnp.transpose(nbr_fea, (2, 1, 0)).reshape(M * NBRF, N_ATOMS)
    cry_flat = crystal_atom_idx.reshape(-1).astype(jnp.int32)

    x_pad = pl.pallas_call(
        _embed_body,
        grid=(1,),
        in_specs=[
            pl.BlockSpec((ORIG, N_ATOMS), lambda i: (0, 0)),
            pl.BlockSpec((AF, ORIG), lambda i: (0, 0)),
        ],
        out_specs=pl.BlockSpec((N_ATOMS, TWO_AF), lambda i: (0, 0)),
        out_shape=jax.ShapeDtypeStruct((N_ATOMS, TWO_AF), jnp.float32),
    )(jnp.transpose(atom_fea), emb_w)

    convs = [
        (fc_w_0, fc_b_0, bn1_g_0, bn1_b_0, bn2_g_0, bn2_b_0),
        (fc_w_1, fc_b_1, bn1_g_1, bn1_b_1, bn2_g_1, bn2_b_1),
        (fc_w_2, fc_b_2, bn1_g_2, bn1_b_2, bn2_g_2, bn2_b_2),
    ]
    eye_m = jnp.eye(M, dtype=jnp.float32)
    for fc_w, fc_b, g1, b1, g2, b2 in convs:
        ws_t = fc_w[:, :AF].T
        wn_t = fc_w[:, AF:TWO_AF].T
        we_t = fc_w[:, TWO_AF:].T
        web = (eye_m[None, :, :, None] * we_t[:, None, None, :]) \
            .reshape(M * NBRF, M * TWO_AF).astype(jnp.bfloat16)
        s, zn = pl.pallas_call(
            _prep_body,
            grid=(1,),
            in_specs=[
                pl.BlockSpec((N_ATOMS, TWO_AF), lambda i: (0, 0)),
                pl.BlockSpec((AF, TWO_AF), lambda i: (0, 0)),
                pl.BlockSpec((AF, TWO_AF), lambda i: (0, 0)),
            ],
            out_specs=[
                pl.BlockSpec((N_ATOMS, TWO_AF), lambda i: (0, 0)),
                pl.BlockSpec((N_ATOMS, TWO_AF), lambda i: (0, 0)),
            ],
            out_shape=[
                jax.ShapeDtypeStruct((N_ATOMS, TWO_AF), jnp.float32),
                jax.ShapeDtypeStruct((N_ATOMS, TWO_AF), jnp.float32),
            ],
        )(x_pad, ws_t, wn_t)
        g = _sc_gather(zn, nbr_idx_flat, window=128)
        g3 = g.reshape(M, N_ATOMS, TWO_AF)
        x_pad = _conv(x_pad, g3, nbr2, s, web, fc_b, g1, b1, g2, b2)

    af = _sc_gather(x_pad, cry_flat, window=128)

    ep6, feat6 = pl.pallas_call(
        _dec_body,
        grid=(1,),
        in_specs=[
            pl.BlockSpec((B * NC, TWO_AF), lambda i: (0, 0)),
            pl.BlockSpec((6, AF, AF), lambda i: (0, 0, 0)),
            pl.BlockSpec((6, 1), lambda i: (0, 0)),
            pl.BlockSpec((6, 6), lambda i: (0, 0)),
            pl.BlockSpec((6, 1), lambda i: (0, 0)),
            pl.BlockSpec((ORIG, AF), lambda i: (0, 0)),
            pl.BlockSpec((ORIG, 1), lambda i: (0, 0)),
        ],
        out_specs=[
            pl.BlockSpec((6, B, NC * NC), lambda i: (0, 0, 0)),
            pl.BlockSpec((ORIG, B, NC), lambda i: (0, 0, 0)),
        ],
        out_shape=[
            jax.ShapeDtypeStruct((6, B, NC * NC), jnp.float32),
            jax.ShapeDtypeStruct((ORIG, B, NC), jnp.float32),
        ],
    )(af, jnp.swapaxes(bil_w, 1, 2), bil_b.reshape(6, 1),
      fc1_w, fc1_b.reshape(6, 1), fcaf_w, fcaf_b.reshape(ORIG, 1))

    return (jnp.transpose(ep6, (1, 2, 0)), jnp.transpose(feat6, (1, 2, 0)))
